# baseline (speedup 1.0000x reference)
import functools

import jax
import jax.numpy as jnp
from jax import lax
from jax.experimental import pallas as pl
from jax.experimental.pallas import tpu as pltpu
from jax.experimental.pallas import tpu_sc as plsc

N_CORES = 2
N_SUBCORES = 16
NW = N_CORES * N_SUBCORES
LANES = 16
NBUF = 4


def _make_degree_kernel(e_total, n_nodes):
    ew = e_total // NW
    n_iter = ew // LANES
    z_iter = n_nodes // LANES

    @functools.partial(
        pl.kernel,
        out_type=jax.ShapeDtypeStruct((NW * n_nodes,), jnp.float32),
        mesh=plsc.VectorSubcoreMesh(core_axis_name="c", subcore_axis_name="s"),
        scratch_types=[
            pltpu.VMEM((ew,), jnp.int32),
            pltpu.VMEM((n_nodes,), jnp.float32),
        ],
        compiler_params=pltpu.CompilerParams(needs_layout_passes=False),
    )
    def deg_kernel(col_hbm, out_hbm, colbuf, degbuf):
        cid = lax.axis_index("c")
        sid = lax.axis_index("s")
        wid = cid * N_SUBCORES + sid
        pltpu.sync_copy(col_hbm.at[pl.ds(wid * ew, ew)], colbuf)
        zeros16 = jnp.zeros((LANES,), jnp.float32)
        ones16 = jnp.ones((LANES,), jnp.float32)

        def zero_body(i, carry):
            degbuf[pl.ds(i * LANES, LANES)] = zeros16
            return carry

        lax.fori_loop(0, z_iter, zero_body, 0)

        def acc_body(i, carry):
            idx = colbuf[pl.ds(i * LANES, LANES)]
            plsc.addupdate_scatter(degbuf, [idx], ones16)
            return carry

        lax.fori_loop(0, n_iter, acc_body, 0)
        pltpu.sync_copy(degbuf, out_hbm.at[pl.ds(wid * n_nodes, n_nodes)])

    return deg_kernel


def _make_agg_kernel(e_total, k, n_pad, d):
    ew = e_total // NW
    n_chunks = ew // k
    n_outer = (n_chunks + NBUF - 1) // NBUF
    slab = n_pad // N_SUBCORES

    @functools.partial(
        pl.kernel,
        out_type=jax.ShapeDtypeStruct((N_CORES, n_pad, d), jnp.float32),
        mesh=plsc.VectorSubcoreMesh(core_axis_name="c", subcore_axis_name="s"),
        scratch_types=[
            pltpu.VMEM((2 * NBUF, k), jnp.int32),
            pltpu.VMEM((2 * NBUF, k), jnp.int32),
            pltpu.VMEM((NBUF, k, d), jnp.float32),
            pltpu.VMEM_SHARED((n_pad, d), jnp.float32),
            pltpu.SemaphoreType.DMA((2 * NBUF,)),
            pltpu.SemaphoreType.DMA((2 * NBUF,)),
            pltpu.SemaphoreType.DMA((NBUF,)),
            pltpu.SemaphoreType.DMA((NBUF,)),
        ],
        compiler_params=pltpu.CompilerParams(needs_layout_passes=False),
    )
    def agg_kernel(row_hbm, col_hbm, xs_hbm, zeros_hbm, out_hbm,
                   iring, cring, xbufs, agg_sh, isem, csem, gsem, ssem):
        cid = lax.axis_index("c")
        sid = lax.axis_index("s")
        wid = cid * N_SUBCORES + sid
        base = wid * ew
        pltpu.sync_copy(zeros_hbm, agg_sh.at[pl.ds(sid * slab, slab)])
        plsc.subcore_barrier()

        IR = 2 * NBUF

        def icopy(j, b):
            return pltpu.make_async_copy(
                row_hbm.at[pl.ds(base + j * k, k)], iring.at[b], isem.at[b])

        def ccopy(j, b):
            return pltpu.make_async_copy(
                col_hbm.at[pl.ds(base + j * k, k)], cring.at[b], csem.at[b])

        def gather(jb, b):
            return pltpu.make_async_copy(
                xs_hbm.at[iring.at[jb]], xbufs.at[b], gsem.at[b])

        def scatter(jb, b):
            return pltpu.make_async_copy(
                xbufs.at[b], agg_sh.at[cring.at[jb]], ssem.at[b])

        for u in range(4):
            icopy(u, u).start()
            ccopy(u, u).start()
        for u in range(2):
            icopy(u, u).wait()
            gather(u, u % NBUF).start()

        UNROLL = 2 * NBUF

        def body(jj, carry):
            for u in range(UNROLL):
                t = jj * UNROLL + u

                @pl.when(t < n_chunks)
                def _():
                    gather(t % IR, t % NBUF).wait()
                    ccopy(t, t % IR).wait()
                    scatter(t % IR, t % NBUF).start(add=True)

                @pl.when(t + 4 < n_chunks)
                def _():
                    icopy(t + 4, (t + 4) % IR).start()
                    ccopy(t + 4, (t + 4) % IR).start()

                @pl.when(t + 2 < n_chunks)
                def _():
                    @pl.when(t >= 2)
                    def _():
                        scatter((t - 2) % IR, (t - 2) % NBUF).wait()
                    icopy(t + 2, (t + 2) % IR).wait()
                    gather((t + 2) % IR, (t + 2) % NBUF).start()

            return carry

        lax.fori_loop(0, (n_chunks + UNROLL - 1) // UNROLL, body, 0)
        for back in (4, 3, 2, 1):
            j = n_chunks - back
            scatter(j % IR, j % NBUF).wait()
        plsc.subcore_barrier()
        pltpu.sync_copy(agg_sh.at[pl.ds(sid * slab, slab)],
                        out_hbm.at[cid, pl.ds(sid * slab, slab)])

    return agg_kernel


def _prep_body(parts_ref, x_ref, dinv_ref, xs_ref):
    parts = parts_ref[...]
    ones = jnp.ones((parts.shape[0], 1), jnp.float32)
    deg = lax.dot_general(parts, ones, (((0,), (0,)), ((), ())),
                          preferred_element_type=jnp.float32) + 1.0
    dinv = lax.rsqrt(deg)
    dinv_ref[...] = dinv
    xs_ref[...] = dinv * x_ref[...]


def _final_body(p0_ref, p1_ref, xs_ref, dinv_ref, w_ref, b_ref, out_ref):
    acc = (p0_ref[...] + p1_ref[...] + xs_ref[...]) * dinv_ref[...]
    out_ref[...] = jnp.dot(acc, w_ref[...],
                           preferred_element_type=jnp.float32) + b_ref[...]


def kernel(x, edge_index, W, b):
    n, d = x.shape
    h = W.shape[1]
    e = edge_index.shape[1]
    ei = edge_index.astype(jnp.int32)
    row, col = ei[0], ei[1]

    k = 80
    n_pad = ((n + 8 * N_SUBCORES - 1) // (8 * N_SUBCORES)) * 8 * N_SUBCORES

    parts_deg = _make_degree_kernel(e, n)(col).reshape(NW, n)

    dinv, xs = pl.pallas_call(
        _prep_body,
        out_shape=(jax.ShapeDtypeStruct((n, 1), jnp.float32),
                   jax.ShapeDtypeStruct((n, d), jnp.float32)),
    )(parts_deg, x)

    zeros_in = jnp.zeros((n_pad // N_SUBCORES, d), jnp.float32)
    parts = _make_agg_kernel(e, k, n_pad, d)(row, col, xs, zeros_in)

    rb = 1000
    out = pl.pallas_call(
        _final_body,
        grid=(n // rb,),
        in_specs=[
            pl.BlockSpec((rb, d), lambda i: (i, 0)),
            pl.BlockSpec((rb, d), lambda i: (i, 0)),
            pl.BlockSpec((rb, d), lambda i: (i, 0)),
            pl.BlockSpec((rb, 1), lambda i: (i, 0)),
            pl.BlockSpec((d, h), lambda i: (0, 0)),
            pl.BlockSpec((1, h), lambda i: (0, 0)),
        ],
        out_specs=pl.BlockSpec((rb, h), lambda i: (i, 0)),
        out_shape=jax.ShapeDtypeStruct((n, h), jnp.float32),
    )(parts[0], parts[1], xs, dinv, W, b.reshape(1, h))
    return out


# speedup vs baseline: 1.0581x; 1.0581x over previous
import functools

import jax
import jax.numpy as jnp
from jax import lax
from jax.experimental import pallas as pl
from jax.experimental.pallas import tpu as pltpu
from jax.experimental.pallas import tpu_sc as plsc

N_CORES = 2
N_SUBCORES = 16
NW = N_CORES * N_SUBCORES
LANES = 16
NBUF = 4


def _make_degree_kernel(e_total, n_nodes):
    ew = e_total // NW
    n_iter = ew // LANES
    z_iter = n_nodes // LANES

    @functools.partial(
        pl.kernel,
        out_type=jax.ShapeDtypeStruct((NW * n_nodes,), jnp.float32),
        mesh=plsc.VectorSubcoreMesh(core_axis_name="c", subcore_axis_name="s"),
        scratch_types=[
            pltpu.VMEM((ew,), jnp.int32),
            pltpu.VMEM((n_nodes,), jnp.float32),
        ],
        compiler_params=pltpu.CompilerParams(needs_layout_passes=False),
    )
    def deg_kernel(col_hbm, out_hbm, colbuf, degbuf):
        cid = lax.axis_index("c")
        sid = lax.axis_index("s")
        wid = cid * N_SUBCORES + sid
        pltpu.sync_copy(col_hbm.at[pl.ds(wid * ew, ew)], colbuf)
        zeros16 = jnp.zeros((LANES,), jnp.float32)
        ones16 = jnp.ones((LANES,), jnp.float32)

        def zero_body(i, carry):
            degbuf[pl.ds(i * LANES, LANES)] = zeros16
            return carry

        lax.fori_loop(0, z_iter, zero_body, 0)

        def acc_body(i, carry):
            idx = colbuf[pl.ds(i * LANES, LANES)]
            plsc.addupdate_scatter(degbuf, [idx], ones16)
            return carry

        lax.fori_loop(0, n_iter, acc_body, 0)
        pltpu.sync_copy(degbuf, out_hbm.at[pl.ds(wid * n_nodes, n_nodes)])

    return deg_kernel


def _make_agg_kernel(e_total, k, n_pad, d):
    ew = e_total // NW
    n_chunks = ew // k
    n_outer = (n_chunks + NBUF - 1) // NBUF
    slab = n_pad // N_SUBCORES

    @functools.partial(
        pl.kernel,
        out_type=jax.ShapeDtypeStruct((N_CORES, n_pad, d), jnp.float32),
        mesh=plsc.VectorSubcoreMesh(core_axis_name="c", subcore_axis_name="s"),
        scratch_types=[
            pltpu.VMEM((NBUF, k), jnp.int32),
            pltpu.VMEM((NBUF, k), jnp.int32),
            pltpu.VMEM((NBUF, k, d), jnp.float32),
            pltpu.VMEM_SHARED((n_pad, d), jnp.float32),
            pltpu.SemaphoreType.DMA((NBUF,)),
            pltpu.SemaphoreType.DMA((NBUF,)),
            pltpu.SemaphoreType.DMA((NBUF,)),
            pltpu.SemaphoreType.DMA((NBUF,)),
        ],
        compiler_params=pltpu.CompilerParams(needs_layout_passes=False),
    )
    def agg_kernel(row_hbm, col_hbm, xs_hbm, zeros_hbm, out_hbm,
                   iring, cring, xbufs, agg_sh, isem, csem, gsem, ssem):
        cid = lax.axis_index("c")
        sid = lax.axis_index("s")
        wid = cid * N_SUBCORES + sid
        base = wid * ew
        pltpu.sync_copy(zeros_hbm, agg_sh.at[pl.ds(sid * slab, slab)])
        plsc.subcore_barrier()

        def icopy(j, b):
            return pltpu.make_async_copy(
                row_hbm.at[pl.ds(base + j * k, k)], iring.at[b], isem.at[b])

        def ccopy(j, b):
            return pltpu.make_async_copy(
                col_hbm.at[pl.ds(base + j * k, k)], cring.at[b], csem.at[b])

        def gather(b):
            return pltpu.make_async_copy(
                xs_hbm.at[iring.at[b]], xbufs.at[b], gsem.at[b])

        def scatter(b):
            return pltpu.make_async_copy(
                xbufs.at[b], agg_sh.at[cring.at[b]], ssem.at[b])

        for b in range(NBUF):
            icopy(b, b).start()
            ccopy(b, b).start()
        for b in range(NBUF):
            icopy(b, b).wait()
            gather(b).start()

        def body(jj, carry):
            for b in range(NBUF):
                j = jj * NBUF + b

                @pl.when(j < n_chunks)
                def _():
                    gather(b).wait()
                    ccopy(j, b).wait()
                    scatter(b).start(add=True)
                    nxt = j + NBUF

                    @pl.when(nxt < n_chunks)
                    def _():
                        icopy(nxt, b).start()
                        scatter(b).wait()
                        ccopy(nxt, b).start()
                        icopy(nxt, b).wait()
                        gather(b).start()

            return carry

        lax.fori_loop(0, (n_chunks + NBUF - 1) // NBUF, body, 0)
        for b in range(NBUF):
            scatter(b).wait()
        plsc.subcore_barrier()
        pltpu.sync_copy(agg_sh.at[pl.ds(sid * slab, slab)],
                        out_hbm.at[cid, pl.ds(sid * slab, slab)])

    return agg_kernel


def _prep_body(parts_ref, x_ref, dinv_ref, xs_ref):
    parts = parts_ref[...]
    ones = jnp.ones((parts.shape[0], 1), jnp.float32)
    deg = lax.dot_general(parts, ones, (((0,), (0,)), ((), ())),
                          preferred_element_type=jnp.float32) + 1.0
    dinv = lax.rsqrt(deg)
    dinv_ref[...] = dinv
    xs_ref[...] = dinv * x_ref[...]


def _selfterm_body(xs_ref, dinv_ref, w_ref, b_ref, out_ref):
    acc = xs_ref[...] * dinv_ref[...]
    out_ref[...] = jnp.dot(acc, w_ref[...],
                           preferred_element_type=jnp.float32) + b_ref[...]


def _final_body(p0_ref, p1_ref, dinv_ref, w_ref, base_ref, out_ref):
    acc = (p0_ref[...] + p1_ref[...]) * dinv_ref[...]
    out_ref[...] = jnp.dot(acc, w_ref[...],
                           preferred_element_type=jnp.float32) + base_ref[...]


def kernel(x, edge_index, W, b):
    n, d = x.shape
    h = W.shape[1]
    e = edge_index.shape[1]
    ei = edge_index.astype(jnp.int32)
    row, col = ei[0], ei[1]

    k = 80
    n_pad = ((n + 8 * N_SUBCORES - 1) // (8 * N_SUBCORES)) * 8 * N_SUBCORES

    parts_deg = _make_degree_kernel(e, n)(col).reshape(NW, n)

    dinv, xs = pl.pallas_call(
        _prep_body,
        out_shape=(jax.ShapeDtypeStruct((n, 1), jnp.float32),
                   jax.ShapeDtypeStruct((n, d), jnp.float32)),
    )(parts_deg, x)

    zeros_in = jnp.zeros((n_pad // N_SUBCORES, d), jnp.float32)
    parts = _make_agg_kernel(e, k, n_pad, d)(row, col, xs, zeros_in)

    rb = 1000
    base_out = pl.pallas_call(
        _selfterm_body,
        grid=(n // rb,),
        in_specs=[
            pl.BlockSpec((rb, d), lambda i: (i, 0)),
            pl.BlockSpec((rb, 1), lambda i: (i, 0)),
            pl.BlockSpec((d, h), lambda i: (0, 0)),
            pl.BlockSpec((1, h), lambda i: (0, 0)),
        ],
        out_specs=pl.BlockSpec((rb, h), lambda i: (i, 0)),
        out_shape=jax.ShapeDtypeStruct((n, h), jnp.float32),
    )(xs, dinv, W, b.reshape(1, h))

    out = pl.pallas_call(
        _final_body,
        grid=(n // rb,),
        in_specs=[
            pl.BlockSpec((rb, d), lambda i: (i, 0)),
            pl.BlockSpec((rb, d), lambda i: (i, 0)),
            pl.BlockSpec((rb, 1), lambda i: (i, 0)),
            pl.BlockSpec((d, h), lambda i: (0, 0)),
            pl.BlockSpec((rb, h), lambda i: (i, 0)),
        ],
        out_specs=pl.BlockSpec((rb, h), lambda i: (i, 0)),
        out_shape=jax.ShapeDtypeStruct((n, h), jnp.float32),
    )(parts[0], parts[1], dinv, W, base_out)
    return out


# speedup vs baseline: 1.1267x; 1.0649x over previous
import functools

import jax
import jax.numpy as jnp
from jax import lax
from jax.experimental import pallas as pl
from jax.experimental.pallas import tpu as pltpu
from jax.experimental.pallas import tpu_sc as plsc

N_CORES = 2
N_SUBCORES = 16
NW = N_CORES * N_SUBCORES
LANES = 16
NBUF = 4


def _make_degree_kernel(e_total, n_nodes):
    ew = e_total // NW
    n_iter = ew // LANES
    z_iter = n_nodes // LANES

    @functools.partial(
        pl.kernel,
        out_type=jax.ShapeDtypeStruct((NW * n_nodes,), jnp.float32),
        mesh=plsc.VectorSubcoreMesh(core_axis_name="c", subcore_axis_name="s"),
        scratch_types=[
            pltpu.VMEM((ew,), jnp.int32),
            pltpu.VMEM((n_nodes,), jnp.float32),
        ],
        compiler_params=pltpu.CompilerParams(needs_layout_passes=False),
    )
    def deg_kernel(col_hbm, out_hbm, colbuf, degbuf):
        cid = lax.axis_index("c")
        sid = lax.axis_index("s")
        wid = cid * N_SUBCORES + sid
        pltpu.sync_copy(col_hbm.at[pl.ds(wid * ew, ew)], colbuf)
        zeros16 = jnp.zeros((LANES,), jnp.float32)
        ones16 = jnp.ones((LANES,), jnp.float32)

        def zero_body(i, carry):
            degbuf[pl.ds(i * LANES, LANES)] = zeros16
            return carry

        lax.fori_loop(0, z_iter, zero_body, 0)

        def acc_body(i, carry):
            idx = colbuf[pl.ds(i * LANES, LANES)]
            plsc.addupdate_scatter(degbuf, [idx], ones16)
            return carry

        lax.fori_loop(0, n_iter, acc_body, 0)
        pltpu.sync_copy(degbuf, out_hbm.at[pl.ds(wid * n_nodes, n_nodes)])

    return deg_kernel


def _make_agg_kernel(e_total, k, n_pad, d):
    ew = e_total // NW
    n_chunks = ew // k
    n_outer = (n_chunks + NBUF - 1) // NBUF
    slab = n_pad // N_SUBCORES

    @functools.partial(
        pl.kernel,
        out_type=jax.ShapeDtypeStruct((N_CORES, n_pad, d), jnp.float32),
        mesh=plsc.VectorSubcoreMesh(core_axis_name="c", subcore_axis_name="s"),
        scratch_types=[
            pltpu.VMEM((NBUF, k), jnp.int32),
            pltpu.VMEM((NBUF, k), jnp.int32),
            pltpu.VMEM((NBUF, k, d), jnp.float32),
            pltpu.VMEM_SHARED((n_pad, d), jnp.float32),
            pltpu.SemaphoreType.DMA((NBUF,)),
            pltpu.SemaphoreType.DMA((NBUF,)),
            pltpu.SemaphoreType.DMA((NBUF,)),
            pltpu.SemaphoreType.DMA((NBUF,)),
        ],
        compiler_params=pltpu.CompilerParams(needs_layout_passes=False),
    )
    def agg_kernel(row_hbm, col_hbm, xs_hbm, zeros_hbm, out_hbm,
                   iring, cring, xbufs, agg_sh, isem, csem, gsem, ssem):
        cid = lax.axis_index("c")
        sid = lax.axis_index("s")
        wid = cid * N_SUBCORES + sid
        base = wid * ew
        pltpu.sync_copy(zeros_hbm, agg_sh.at[pl.ds(sid * slab, slab)])
        plsc.subcore_barrier()

        def icopy(j, b):
            return pltpu.make_async_copy(
                row_hbm.at[pl.ds(base + j * k, k)], iring.at[b], isem.at[b])

        def ccopy(j, b):
            return pltpu.make_async_copy(
                col_hbm.at[pl.ds(base + j * k, k)], cring.at[b], csem.at[b])

        def gather(b):
            return pltpu.make_async_copy(
                xs_hbm.at[iring.at[b]], xbufs.at[b], gsem.at[b])

        def scatter(b):
            return pltpu.make_async_copy(
                xbufs.at[b], agg_sh.at[cring.at[b]], ssem.at[b])

        for b in range(NBUF):
            icopy(b, b).start()
            ccopy(b, b).start()
        for b in range(NBUF):
            icopy(b, b).wait()
            gather(b).start()

        def body(jj, carry):
            for b in range(NBUF):
                j = jj * NBUF + b

                @pl.when(j < n_chunks)
                def _():
                    gather(b).wait()
                    ccopy(j, b).wait()
                    scatter(b).start(add=True)
                    nxt = j + NBUF

                    @pl.when(nxt < n_chunks)
                    def _():
                        icopy(nxt, b).start()
                        scatter(b).wait()
                        ccopy(nxt, b).start()
                        icopy(nxt, b).wait()
                        gather(b).start()

            return carry

        lax.fori_loop(0, (n_chunks + NBUF - 1) // NBUF, body, 0)
        for b in range(NBUF):
            scatter(b).wait()
        plsc.subcore_barrier()
        pltpu.sync_copy(agg_sh.at[pl.ds(sid * slab, slab)],
                        out_hbm.at[cid, pl.ds(sid * slab, slab)])

    return agg_kernel


def _prep_body(parts_ref, x_ref, dinv_ref, xs_ref):
    parts = parts_ref[...]
    ones = jnp.ones((parts.shape[0], 1), jnp.float32)
    deg = lax.dot_general(parts, ones, (((0,), (0,)), ((), ())),
                          preferred_element_type=jnp.float32) + 1.0
    dinv = lax.rsqrt(deg)
    dinv_ref[...] = dinv
    xs_ref[...] = dinv * x_ref[...]


def _final_body(p0_ref, p1_ref, xs_ref, dinv_ref, w_ref, b_ref, out_ref):
    acc = (p0_ref[...] + p1_ref[...] + xs_ref[...]) * dinv_ref[...]
    out_ref[...] = jnp.dot(acc, w_ref[...],
                           preferred_element_type=jnp.float32) + b_ref[...]


def kernel(x, edge_index, W, b):
    n, d = x.shape
    h = W.shape[1]
    e = edge_index.shape[1]
    ei = edge_index.astype(jnp.int32)
    row, col = ei[0], ei[1]

    k = 80
    n_pad = ((n + 8 * N_SUBCORES - 1) // (8 * N_SUBCORES)) * 8 * N_SUBCORES

    parts_deg = _make_degree_kernel(e, n)(col).reshape(NW, n)

    dinv, xs = pl.pallas_call(
        _prep_body,
        out_shape=(jax.ShapeDtypeStruct((n, 1), jnp.float32),
                   jax.ShapeDtypeStruct((n, d), jnp.float32)),
    )(parts_deg, x)

    zeros_in = jnp.zeros((n_pad // N_SUBCORES, d), jnp.float32)
    parts = _make_agg_kernel(e, k, n_pad, d)(row, col, xs, zeros_in)

    rb = 1000
    out = pl.pallas_call(
        _final_body,
        grid=(n // rb,),
        in_specs=[
            pl.BlockSpec((rb, d), lambda i: (i, 0)),
            pl.BlockSpec((rb, d), lambda i: (i, 0)),
            pl.BlockSpec((rb, d), lambda i: (i, 0)),
            pl.BlockSpec((rb, 1), lambda i: (i, 0)),
            pl.BlockSpec((d, h), lambda i: (0, 0)),
            pl.BlockSpec((1, h), lambda i: (0, 0)),
        ],
        out_specs=pl.BlockSpec((rb, h), lambda i: (i, 0)),
        out_shape=jax.ShapeDtypeStruct((n, h), jnp.float32),
    )(parts[0], parts[1], xs, dinv, W, b.reshape(1, h))
    return out
